# initial kernel scaffold (unmeasured)
import jax
import jax.numpy as jnp
from jax import lax
from jax.experimental import pallas as pl
from jax.experimental.pallas import tpu as pltpu

N_DEV = 4


def kernel(x, w_mat):
    m_per, k = x.shape
    _, n = w_mat.shape
    n_per = n // N_DEV

    def body(x_ref, w_ref, out_ref, y_send, send_sems, recv_sems):
        my = lax.axis_index("i")

        barrier_sem = pltpu.get_barrier_semaphore()
        for j in range(N_DEV):
            @pl.when(my != j)
            def _():
                pl.semaphore_signal(
                    barrier_sem, inc=1, device_id=(j,),
                    device_id_type=pl.DeviceIdType.MESH,
                )
        pl.semaphore_wait(barrier_sem, N_DEV - 1)

        for j in range(N_DEV):
            y = jnp.maximum(
                jnp.dot(
                    x_ref[...],
                    w_ref[:, j * n_per:(j + 1) * n_per],
                    preferred_element_type=jnp.float32,
                ),
                0.0,
            )

            @pl.when(my == j)
            def _():
                out_ref[j * m_per:(j + 1) * m_per, :] = y

            @pl.when(my != j)
            def _():
                y_send[j] = y
                rdma = pltpu.make_async_remote_copy(
                    src_ref=y_send.at[j],
                    dst_ref=out_ref.at[pl.ds(my * m_per, m_per)],
                    send_sem=send_sems.at[j],
                    recv_sem=recv_sems.at[my],
                    device_id=(j,),
                    device_id_type=pl.DeviceIdType.MESH,
                )
                rdma.start()

        for p in range(N_DEV):
            @pl.when(my != p)
            def _():
                recv = pltpu.make_async_remote_copy(
                    src_ref=y_send.at[p],
                    dst_ref=out_ref.at[pl.ds(p * m_per, m_per)],
                    send_sem=send_sems.at[p],
                    recv_sem=recv_sems.at[p],
                    device_id=(p,),
                    device_id_type=pl.DeviceIdType.MESH,
                )
                recv.wait_recv()

        for j in range(N_DEV):
            @pl.when(my != j)
            def _():
                snd = pltpu.make_async_remote_copy(
                    src_ref=y_send.at[j],
                    dst_ref=out_ref.at[pl.ds(my * m_per, m_per)],
                    send_sem=send_sems.at[j],
                    recv_sem=recv_sems.at[j],
                    device_id=(j,),
                    device_id_type=pl.DeviceIdType.MESH,
                )
                snd.wait_send()

    return pl.pallas_call(
        body,
        out_shape=jax.ShapeDtypeStruct((N_DEV * m_per, n_per), jnp.float32),
        in_specs=[
            pl.BlockSpec(memory_space=pltpu.VMEM),
            pl.BlockSpec(memory_space=pltpu.VMEM),
        ],
        out_specs=pl.BlockSpec(memory_space=pltpu.VMEM),
        scratch_shapes=[
            pltpu.VMEM((N_DEV, m_per, n_per), jnp.float32),
            pltpu.SemaphoreType.DMA((N_DEV,)),
            pltpu.SemaphoreType.DMA((N_DEV,)),
        ],
        compiler_params=pltpu.CompilerParams(collective_id=0),
    )(x, w_mat)


# baseline (device time: 86221 ns/iter reference)
import jax
import jax.numpy as jnp
from jax import lax
from jax.experimental import pallas as pl
from jax.experimental.pallas import tpu as pltpu

N_DEV = 4


def kernel(x, w_mat):
    m_per, k = x.shape
    _, n = w_mat.shape
    n_per = n // N_DEV

    def w_copy(w_ref, w_buf, copy_sems, j):
        return pltpu.make_async_copy(
            w_ref.at[:, pl.ds(j * n_per, n_per)],
            w_buf.at[j % 2],
            copy_sems.at[j % 2],
        )

    def body(x_ref, w_ref, out_ref, w_buf, y_send, copy_sems, send_sems,
             recv_sems):
        my = lax.axis_index("i")

        w_copy(w_ref, w_buf, copy_sems, 0).start()
        w_copy(w_ref, w_buf, copy_sems, 1).start()

        barrier_sem = pltpu.get_barrier_semaphore()
        for j in range(N_DEV):
            @pl.when(my != j)
            def _():
                pl.semaphore_signal(
                    barrier_sem, inc=1, device_id=(j,),
                    device_id_type=pl.DeviceIdType.MESH,
                )
        pl.semaphore_wait(barrier_sem, N_DEV - 1)

        for j in range(N_DEV):
            w_copy(w_ref, w_buf, copy_sems, j).wait()
            y = jnp.maximum(
                jnp.dot(
                    x_ref[...],
                    w_buf[j % 2],
                    preferred_element_type=jnp.float32,
                ),
                0.0,
            )

            @pl.when(my == j)
            def _():
                out_ref[j * m_per:(j + 1) * m_per, :] = y

            @pl.when(my != j)
            def _():
                y_send[j] = y
                rdma = pltpu.make_async_remote_copy(
                    src_ref=y_send.at[j],
                    dst_ref=out_ref.at[pl.ds(my * m_per, m_per)],
                    send_sem=send_sems.at[j],
                    recv_sem=recv_sems.at[my],
                    device_id=(j,),
                    device_id_type=pl.DeviceIdType.MESH,
                )
                rdma.start()

            if j + 2 < N_DEV:
                w_copy(w_ref, w_buf, copy_sems, j + 2).start()

        for p in range(N_DEV):
            @pl.when(my != p)
            def _():
                recv = pltpu.make_async_remote_copy(
                    src_ref=y_send.at[p],
                    dst_ref=out_ref.at[pl.ds(p * m_per, m_per)],
                    send_sem=send_sems.at[p],
                    recv_sem=recv_sems.at[p],
                    device_id=(p,),
                    device_id_type=pl.DeviceIdType.MESH,
                )
                recv.wait_recv()

        for j in range(N_DEV):
            @pl.when(my != j)
            def _():
                snd = pltpu.make_async_remote_copy(
                    src_ref=y_send.at[j],
                    dst_ref=out_ref.at[pl.ds(my * m_per, m_per)],
                    send_sem=send_sems.at[j],
                    recv_sem=recv_sems.at[j],
                    device_id=(j,),
                    device_id_type=pl.DeviceIdType.MESH,
                )
                snd.wait_send()

    return pl.pallas_call(
        body,
        out_shape=jax.ShapeDtypeStruct((N_DEV * m_per, n_per), jnp.float32),
        in_specs=[
            pl.BlockSpec(memory_space=pltpu.VMEM),
            pl.BlockSpec(memory_space=pl.ANY),
        ],
        out_specs=pl.BlockSpec(memory_space=pltpu.VMEM),
        scratch_shapes=[
            pltpu.VMEM((2, k, n_per), jnp.float32),
            pltpu.VMEM((N_DEV, m_per, n_per), jnp.float32),
            pltpu.SemaphoreType.DMA((2,)),
            pltpu.SemaphoreType.DMA((N_DEV,)),
            pltpu.SemaphoreType.DMA((N_DEV,)),
        ],
        compiler_params=pltpu.CompilerParams(
            collective_id=0,
            vmem_limit_bytes=100 * 1024 * 1024,
        ),
    )(x, w_mat)


# device time: 49280 ns/iter; 1.7496x vs baseline; 1.7496x over previous
import jax
import jax.numpy as jnp
from jax import lax
from jax.experimental import pallas as pl
from jax.experimental.pallas import tpu as pltpu

N_DEV = 4
N_ROWS = 2

_OFFS = (1, 2, 3, 0)


def kernel(x, w_mat):
    m_per, k = x.shape
    _, n = w_mat.shape
    n_per = n // N_DEV
    m_half = m_per // N_ROWS
    n_out_dma = N_DEV * N_ROWS

    def body(x_hbm, w_hbm, out_hbm, x_vm, w_buf, y_send, y_recv, f_stg,
             x_sems, pre_sems, copy_sems, out_sems, send_sems, recv_sems):
        my = lax.axis_index("i")

        def dest(off):
            return lax.rem(my + off, N_DEV)

        def x_copy(r):
            return pltpu.make_async_copy(
                x_hbm.at[pl.ds(r * m_half, m_half)],
                x_vm.at[r],
                x_sems.at[r],
            )

        def w_copy(off, slot):
            return pltpu.make_async_copy(
                w_hbm.at[:, pl.ds(dest(off) * n_per, n_per)],
                w_buf.at[slot],
                copy_sems.at[slot],
            )

        def rdma(off, r, dev):
            idx = (off - 1) * N_ROWS + r
            return pltpu.make_async_remote_copy(
                src_ref=y_send.at[idx],
                dst_ref=y_recv.at[idx],
                send_sem=send_sems.at[idx],
                recv_sem=recv_sems.at[idx],
                device_id=(dev,),
                device_id_type=pl.DeviceIdType.MESH,
            )

        out_copies = []

        def out_push(row_start):
            s = len(out_copies)
            slot = s % 4
            if s >= 4:
                out_copies[s - 4].wait()
            cp = pltpu.make_async_copy(
                f_stg.at[slot],
                out_hbm.at[pl.ds(row_start, m_half)],
                out_sems.at[s],
            )
            out_copies.append(cp)
            return slot, cp

        k_half = k // 2

        x0_lo = pltpu.make_async_copy(
            x_hbm.at[pl.ds(0, m_half), pl.ds(0, k_half)],
            x_vm.at[0, :, pl.ds(0, k_half)], pre_sems.at[0])
        w0_lo = pltpu.make_async_copy(
            w_hbm.at[pl.ds(0, k_half), pl.ds(dest(1) * n_per, n_per)],
            w_buf.at[0, pl.ds(0, k_half)], pre_sems.at[1])
        x0_hi = pltpu.make_async_copy(
            x_hbm.at[pl.ds(0, m_half), pl.ds(k_half, k_half)],
            x_vm.at[0, :, pl.ds(k_half, k_half)], pre_sems.at[2])
        w0_hi = pltpu.make_async_copy(
            w_hbm.at[pl.ds(k_half, k_half), pl.ds(dest(1) * n_per, n_per)],
            w_buf.at[0, pl.ds(k_half, k_half)], pre_sems.at[3])
        x0_lo.start()
        w0_lo.start()
        x0_hi.start()
        w0_hi.start()
        for r in range(1, N_ROWS):
            x_copy(r).start()
        w_copy(_OFFS[1], 1).start()

        barrier_sem = pltpu.get_barrier_semaphore()
        for off in range(1, N_DEV):
            pl.semaphore_signal(
                barrier_sem, inc=1, device_id=(dest(off),),
                device_id_type=pl.DeviceIdType.MESH,
            )
        pl.semaphore_wait(barrier_sem, N_DEV - 1)

        def recv_chunk(off):
            src = lax.rem(my - off + N_DEV, N_DEV)
            for r in range(N_ROWS):
                rdma(off, r, src).wait_recv()
                stg, cp = out_push(src * m_per + r * m_half)
                f_stg[stg] = y_recv[(off - 1) * N_ROWS + r].astype(
                    jnp.float32)
                cp.start()

        x0_lo.wait()
        w0_lo.wait()
        acc = jnp.dot(x_vm[0, :, :k_half], w_buf[0, :k_half],
                      preferred_element_type=jnp.float32)
        x0_hi.wait()
        w0_hi.wait()
        y = jnp.maximum(
            acc + jnp.dot(x_vm[0, :, k_half:], w_buf[0, k_half:],
                          preferred_element_type=jnp.float32),
            0.0,
        )
        y_send[0] = y.astype(jnp.bfloat16)
        rdma(1, 0, dest(1)).start()

        x_copy(1).wait()
        y = jnp.maximum(
            jnp.dot(x_vm[1], w_buf[0], preferred_element_type=jnp.float32),
            0.0,
        )
        y_send[1] = y.astype(jnp.bfloat16)
        rdma(1, 1, dest(1)).start()
        w_copy(_OFFS[2], 0).start()

        for step, off in enumerate(_OFFS[1:-1], start=1):
            slot = step % 2
            w_copy(off, slot).wait()
            for r in range(N_ROWS):
                y = jnp.maximum(
                    jnp.dot(
                        x_vm[r],
                        w_buf[slot],
                        preferred_element_type=jnp.float32,
                    ),
                    0.0,
                )
                y_send[(off - 1) * N_ROWS + r] = y.astype(jnp.bfloat16)
                rdma(off, r, dest(off)).start()

            if step + 2 < N_DEV:
                w_copy(_OFFS[step + 2], slot).start()

        w_copy(0, 1).wait()
        for r in range(N_ROWS):
            y = jnp.maximum(
                jnp.dot(
                    x_vm[r],
                    w_buf[1],
                    preferred_element_type=jnp.float32,
                ),
                0.0,
            )
            stg, cp = out_push(my * m_per + r * m_half)
            f_stg[stg] = y
            cp.start()

        for off in range(1, N_DEV):
            recv_chunk(off)

        for cp in out_copies[-4:]:
            cp.wait()
        for off in range(1, N_DEV):
            for r in range(N_ROWS):
                rdma(off, r, dest(off)).wait_send()

    return pl.pallas_call(
        body,
        out_shape=jax.ShapeDtypeStruct((N_DEV * m_per, n_per), jnp.float32),
        in_specs=[
            pl.BlockSpec(memory_space=pl.ANY),
            pl.BlockSpec(memory_space=pl.ANY),
        ],
        out_specs=pl.BlockSpec(memory_space=pl.ANY),
        scratch_shapes=[
            pltpu.VMEM((N_ROWS, m_half, k), jnp.float32),
            pltpu.VMEM((2, k, n_per), jnp.float32),
            pltpu.VMEM(((N_DEV - 1) * N_ROWS, m_half, n_per),
                       jnp.bfloat16),
            pltpu.VMEM(((N_DEV - 1) * N_ROWS, m_half, n_per),
                       jnp.bfloat16),
            pltpu.VMEM((4, m_half, n_per), jnp.float32),
            pltpu.SemaphoreType.DMA((N_ROWS,)),
            pltpu.SemaphoreType.DMA((4,)),
            pltpu.SemaphoreType.DMA((2,)),
            pltpu.SemaphoreType.DMA((N_DEV * N_ROWS,)),
            pltpu.SemaphoreType.DMA(((N_DEV - 1) * N_ROWS,)),
            pltpu.SemaphoreType.DMA(((N_DEV - 1) * N_ROWS,)),
        ],
        compiler_params=pltpu.CompilerParams(
            collective_id=0,
            vmem_limit_bytes=100 * 1024 * 1024,
        ),
    )(x, w_mat)
